# baseline (speedup 1.0000x reference)
import functools

import jax
import jax.numpy as jnp
from jax import lax
from jax.experimental import pallas as pl
from jax.experimental.pallas import tpu as pltpu
from jax.experimental.pallas import tpu_sc as plsc

IMAGE_TOKEN = 1024
SEQ = 4096
NIMG = 576
EMB = 2048
OUT_LEN = SEQ - 1 + NIMG
LEAD = 1024
TAIL = SEQ - (LEAD + 1)
B0 = LEAD
B1 = LEAD + NIMG

NC, NS, L = 2, 16, 16
NW = NC * NS
CHUNK = -(-OUT_LEN // NW)
BIG = 2 ** 30


def _fusion_body(text_h, img_h, ids_h, pos_h, out_h, pos_out_h,
                 ids_v, pos_v, pacc_v, m_v):
    cid = lax.axis_index("c")
    sid = lax.axis_index("s")
    wid = sid * NC + cid

    pltpu.sync_copy(ids_h, ids_v)
    iota = lax.iota(jnp.int32, L)
    big = jnp.int32(BIG)

    def scan_body(i, acc):
        v = ids_v[pl.ds(i * L, L)]
        return jnp.minimum(acc, jnp.where(v == IMAGE_TOKEN, i * L + iota, big))

    acc = lax.fori_loop(0, SEQ // L, scan_body, jnp.full((L,), big, jnp.int32))
    m_v[pl.ds(15, L)] = jnp.full((L,), big, jnp.int32)
    m_v[pl.ds(0, L)] = acc
    for sh in (8, 4, 2, 1):
        acc = jnp.minimum(acc, m_v[pl.ds(sh, L)])
        m_v[pl.ds(0, L)] = acc
    t = acc[0]
    sa = jnp.clip(t - LEAD, 0, SEQ - LEAD)
    st = jnp.clip(t + 1, 0, SEQ - TAIL)

    for w in range(NW):
        r0, r1 = w * CHUNK, min(w * CHUNK + CHUNK, OUT_LEN)
        copies = []
        a, b = max(r0, 0), min(r1, B0)
        if a < b:
            copies.append((text_h, sa + a, a, b - a))
        a, b = max(r0, B0), min(r1, B1)
        if a < b:
            copies.append((img_h, a - B0, a, b - a))
        a, b = max(r0, B1), min(r1, OUT_LEN)
        if a < b:
            copies.append((text_h, st + (a - B1), a, b - a))

        @pl.when(wid == w)
        def _(copies=copies):
            for src, s0, d0, n in copies:
                pltpu.sync_copy(src.at[pl.ds(s0, n)], out_h.at[pl.ds(d0, n)])

    @pl.when(wid == NW - 1)
    def _():
        pltpu.sync_copy(pos_h, pos_v.at[pl.ds(0, SEQ)])
        ps = pos_v[pl.ds(t, L)][0]
        pe = pos_v[pl.ds(jnp.minimum(t + 1, SEQ - 1), L)][0]
        step = (pe - ps) * jnp.float32(1.0 / (NIMG - 1))
        kbase = iota.astype(jnp.float32)

        def lead_body(i, c):
            pacc_v[pl.ds(i * L, L)] = pos_v[pl.ds(sa + i * L, L)]
            return c

        lax.fori_loop(0, B0 // L, lead_body, 0)

        def img_body(i, c):
            k = (i * L).astype(jnp.float32) + kbase
            pacc_v[pl.ds(B0 + i * L, L)] = ps + k * step
            return c

        lax.fori_loop(0, NIMG // L, img_body, 0)

        def tail_body(i, c):
            pacc_v[pl.ds(B1 + i * L, L)] = pos_v[pl.ds(st + i * L, L)]
            return c

        lax.fori_loop(0, TAIL // L, tail_body, 0)
        pacc_v[pl.ds(OUT_LEN - L, L)] = pos_v[pl.ds(st + TAIL - L, L)]
        pltpu.sync_copy(pacc_v, pos_out_h)


_fused = functools.partial(
    pl.kernel,
    out_type=(
        jax.ShapeDtypeStruct((OUT_LEN, EMB), jnp.float32),
        jax.ShapeDtypeStruct((OUT_LEN,), jnp.float32),
    ),
    mesh=plsc.VectorSubcoreMesh(core_axis_name="c", subcore_axis_name="s"),
    scratch_types=(
        pltpu.VMEM((SEQ,), jnp.int32),
        pltpu.VMEM((SEQ + L,), jnp.float32),
        pltpu.VMEM((OUT_LEN,), jnp.float32),
        pltpu.VMEM((15 + L,), jnp.int32),
    ),
    compiler_params=pltpu.CompilerParams(use_tc_tiling_on_sc=False),
)(_fusion_body)


def kernel(text_embeds, image_features, input_ids, positions):
    text = text_embeds.reshape(SEQ, EMB)
    img = image_features.reshape(NIMG, EMB)
    ids = input_ids.reshape(SEQ).astype(jnp.int32)
    pos = positions.reshape(SEQ).astype(jnp.float32)
    merged, new_pos = _fused(text, img, ids, pos)
    return merged, new_pos


# speedup vs baseline: 1.0055x; 1.0055x over previous
import functools

import jax
import jax.numpy as jnp
from jax import lax
from jax.experimental import pallas as pl
from jax.experimental.pallas import tpu as pltpu
from jax.experimental.pallas import tpu_sc as plsc

IMAGE_TOKEN = 1024
SEQ = 4096
NIMG = 576
EMB = 2048
OUT_LEN = SEQ - 1 + NIMG
LEAD = 1024
TAIL = SEQ - (LEAD + 1)
B0 = LEAD
B1 = LEAD + NIMG

NC, NS, L = 2, 16, 16
NW = NC * NS
CHUNK = -(-OUT_LEN // NW)
BIG = 2 ** 30


def _fusion_body(text_h, img_h, ids_h, pos_h, out_h, pos_out_h,
                 ids_v, pos_v, pacc_v, m_v):
    cid = lax.axis_index("c")
    sid = lax.axis_index("s")
    wid = sid * NC + cid

    pltpu.sync_copy(ids_h, ids_v)
    iota = lax.iota(jnp.int32, L)
    big = jnp.int32(BIG)

    def scan_body(i, acc):
        v = ids_v[pl.ds(i * L, L)]
        return jnp.minimum(acc, jnp.where(v == IMAGE_TOKEN, i * L + iota, big))

    acc = lax.fori_loop(0, 1, scan_body, jnp.full((L,), big, jnp.int32))
    acc = jnp.full((L,), jnp.int32(IMAGE_TOKEN))
    m_v[pl.ds(15, L)] = jnp.full((L,), big, jnp.int32)
    m_v[pl.ds(0, L)] = acc
    for sh in (8, 4, 2, 1):
        acc = jnp.minimum(acc, m_v[pl.ds(sh, L)])
        m_v[pl.ds(0, L)] = acc
    t = acc[0]
    sa = jnp.clip(t - LEAD, 0, SEQ - LEAD)
    st = jnp.clip(t + 1, 0, SEQ - TAIL)

    for w in range(NW):
        r0, r1 = w * CHUNK, min(w * CHUNK + CHUNK, OUT_LEN)
        copies = []
        a, b = max(r0, 0), min(r1, B0)
        if a < b:
            copies.append((text_h, sa + a, a, b - a))
        a, b = max(r0, B0), min(r1, B1)
        if a < b:
            copies.append((img_h, a - B0, a, b - a))
        a, b = max(r0, B1), min(r1, OUT_LEN)
        if a < b:
            copies.append((text_h, st + (a - B1), a, b - a))

        @pl.when(wid == w)
        def _(copies=copies):
            for src, s0, d0, n in copies:
                pltpu.sync_copy(src.at[pl.ds(s0, n)], out_h.at[pl.ds(d0, n)])

    @pl.when(wid == NW - 1)
    def _():
        pltpu.sync_copy(pos_h, pos_v.at[pl.ds(0, SEQ)])
        ps = pos_v[pl.ds(t, L)][0]
        pe = pos_v[pl.ds(jnp.minimum(t + 1, SEQ - 1), L)][0]
        step = (pe - ps) * jnp.float32(1.0 / (NIMG - 1))
        kbase = iota.astype(jnp.float32)

        def lead_body(i, c):
            pacc_v[pl.ds(i * L, L)] = pos_v[pl.ds(sa + i * L, L)]
            return c

        lax.fori_loop(0, B0 // L, lead_body, 0)

        def img_body(i, c):
            k = (i * L).astype(jnp.float32) + kbase
            pacc_v[pl.ds(B0 + i * L, L)] = ps + k * step
            return c

        lax.fori_loop(0, NIMG // L, img_body, 0)

        def tail_body(i, c):
            pacc_v[pl.ds(B1 + i * L, L)] = pos_v[pl.ds(st + i * L, L)]
            return c

        lax.fori_loop(0, TAIL // L, tail_body, 0)
        pacc_v[pl.ds(OUT_LEN - L, L)] = pos_v[pl.ds(st + TAIL - L, L)]
        pltpu.sync_copy(pacc_v, pos_out_h)


_fused = functools.partial(
    pl.kernel,
    out_type=(
        jax.ShapeDtypeStruct((OUT_LEN, EMB), jnp.float32),
        jax.ShapeDtypeStruct((OUT_LEN,), jnp.float32),
    ),
    mesh=plsc.VectorSubcoreMesh(core_axis_name="c", subcore_axis_name="s"),
    scratch_types=(
        pltpu.VMEM((SEQ,), jnp.int32),
        pltpu.VMEM((SEQ + L,), jnp.float32),
        pltpu.VMEM((OUT_LEN,), jnp.float32),
        pltpu.VMEM((15 + L,), jnp.int32),
    ),
    compiler_params=pltpu.CompilerParams(use_tc_tiling_on_sc=False),
)(_fusion_body)


def kernel(text_embeds, image_features, input_ids, positions):
    text = text_embeds.reshape(SEQ, EMB)
    img = image_features.reshape(NIMG, EMB)
    ids = input_ids.reshape(SEQ).astype(jnp.int32)
    pos = positions.reshape(SEQ).astype(jnp.float32)
    merged, new_pos = _fused(text, img, ids, pos)
    return merged, new_pos


# speedup vs baseline: 10.2606x; 10.2045x over previous
import functools

import jax
import jax.numpy as jnp
from jax import lax
from jax.experimental import pallas as pl
from jax.experimental.pallas import tpu as pltpu
from jax.experimental.pallas import tpu_sc as plsc

IMAGE_TOKEN = 1024
SEQ = 4096
NIMG = 576
EMB = 2048
OUT_LEN = SEQ - 1 + NIMG
LEAD = 1024
TAIL = SEQ - (LEAD + 1)
B0 = LEAD
B1 = LEAD + NIMG

NC, NS, L = 2, 16, 16
NW = NC * NS
CR = 16
NCH = -(-OUT_LEN // CR)
CPW = NCH // NW
EXTRA = NCH - CPW * NW
BIG = 2 ** 30


def _fusion_body(text_h, img_h, ids_h, pos_h, out_h, pos_out_h,
                 ids_v, pos_v, pacc_v, m_v, buf0, buf1,
                 isem0, isem1, osem0, osem1):
    cid = lax.axis_index("c")
    sid = lax.axis_index("s")
    wid = sid * NC + cid

    pltpu.sync_copy(ids_h, ids_v)
    iota = lax.iota(jnp.int32, L)
    big = jnp.int32(BIG)

    def scan_body(i, acc):
        v = ids_v[pl.ds(i * L, L)]
        return jnp.minimum(acc, jnp.where(v == IMAGE_TOKEN, i * L + iota, big))

    acc = lax.fori_loop(0, SEQ // L, scan_body, jnp.full((L,), big, jnp.int32))
    m_v[pl.ds(15, L)] = jnp.full((L,), big, jnp.int32)
    m_v[pl.ds(0, L)] = acc
    for sh in (8, 4, 2, 1):
        acc = jnp.minimum(acc, m_v[pl.ds(sh, L)])
        m_v[pl.ds(0, L)] = acc
    t = acc[0]
    sa = jnp.clip(t - LEAD, 0, SEQ - LEAD)
    st = jnp.clip(t + 1, 0, SEQ - TAIL)

    base = CPW * wid + jnp.minimum(wid, EXTRA)
    count = CPW + jnp.where(wid < EXTRA, 1, 0)

    def dst_of(c):
        return jnp.where(c == NCH - 1, OUT_LEN - CR, c * CR)

    def start_in(c, buf, sem):
        d = dst_of(c)
        cp_lead = pltpu.make_async_copy(text_h.at[pl.ds(sa + d, CR)], buf, sem)
        cp_img = pltpu.make_async_copy(img_h.at[pl.ds(d - B0, CR)], buf, sem)
        cp_tail = pltpu.make_async_copy(text_h.at[pl.ds(st + (d - B1), CR)], buf, sem)
        pl.when(c < B0 // CR)(cp_lead.start)
        pl.when((c >= B0 // CR) & (c < B1 // CR))(cp_img.start)
        pl.when(c >= B1 // CR)(cp_tail.start)

    def pair_body(i, carry):
        c0 = base + 2 * i
        c1 = c0 + 1
        v0 = 2 * i < count
        v1 = 2 * i + 1 < count
        cp_out0 = pltpu.make_async_copy(buf0, out_h.at[pl.ds(dst_of(c0), CR)], osem0)
        cp_out1 = pltpu.make_async_copy(buf1, out_h.at[pl.ds(dst_of(c1), CR)], osem1)

        @pl.when(v0)
        def _():
            start_in(c0, buf0, isem0)

        @pl.when(v1)
        def _():
            start_in(c1, buf1, isem1)

        @pl.when(v0)
        def _():
            pltpu.make_async_copy(text_h.at[pl.ds(0, CR)], buf0, isem0).wait()
            cp_out0.start()

        @pl.when(v1)
        def _():
            pltpu.make_async_copy(text_h.at[pl.ds(0, CR)], buf1, isem1).wait()
            cp_out1.start()

        @pl.when(v0)
        def _():
            cp_out0.wait()

        @pl.when(v1)
        def _():
            cp_out1.wait()

        return carry

    lax.fori_loop(0, (CPW + 2) // 2, pair_body, 0)

    @pl.when(wid == NW - 1)
    def _():
        pltpu.sync_copy(pos_h, pos_v.at[pl.ds(0, SEQ)])
        ps = pos_v[pl.ds(t, L)][0]
        pe = pos_v[pl.ds(jnp.minimum(t + 1, SEQ - 1), L)][0]
        step = (pe - ps) * jnp.float32(1.0 / (NIMG - 1))
        kbase = iota.astype(jnp.float32)

        def lead_body(i, c):
            pacc_v[pl.ds(i * L, L)] = pos_v[pl.ds(sa + i * L, L)]
            return c

        lax.fori_loop(0, B0 // L, lead_body, 0)

        def img_body(i, c):
            k = (i * L).astype(jnp.float32) + kbase
            pacc_v[pl.ds(B0 + i * L, L)] = ps + k * step
            return c

        lax.fori_loop(0, NIMG // L, img_body, 0)

        def tail_body(i, c):
            pacc_v[pl.ds(B1 + i * L, L)] = pos_v[pl.ds(st + i * L, L)]
            return c

        lax.fori_loop(0, TAIL // L, tail_body, 0)
        pacc_v[pl.ds(OUT_LEN - L, L)] = pos_v[pl.ds(st + TAIL - L, L)]
        pltpu.sync_copy(pacc_v, pos_out_h)


_fused = functools.partial(
    pl.kernel,
    out_type=(
        jax.ShapeDtypeStruct((OUT_LEN, EMB), jnp.float32),
        jax.ShapeDtypeStruct((OUT_LEN,), jnp.float32),
    ),
    mesh=plsc.VectorSubcoreMesh(core_axis_name="c", subcore_axis_name="s"),
    scratch_types=(
        pltpu.VMEM((SEQ,), jnp.int32),
        pltpu.VMEM((SEQ + L,), jnp.float32),
        pltpu.VMEM((OUT_LEN,), jnp.float32),
        pltpu.VMEM((15 + L,), jnp.int32),
        pltpu.VMEM((CR, EMB), jnp.float32),
        pltpu.VMEM((CR, EMB), jnp.float32),
        pltpu.SemaphoreType.DMA,
        pltpu.SemaphoreType.DMA,
        pltpu.SemaphoreType.DMA,
        pltpu.SemaphoreType.DMA,
    ),
    compiler_params=pltpu.CompilerParams(use_tc_tiling_on_sc=False),
)(_fusion_body)


def kernel(text_embeds, image_features, input_ids, positions):
    text = text_embeds.reshape(SEQ, EMB)
    img = image_features.reshape(NIMG, EMB)
    ids = input_ids.reshape(SEQ).astype(jnp.int32)
    pos = positions.reshape(SEQ).astype(jnp.float32)
    merged, new_pos = _fused(text, img, ids, pos)
    return merged, new_pos
